# Initial kernel scaffold; baseline (speedup 1.0000x reference)
import jax
import jax.numpy as jnp
from jax.experimental import pallas as pl


def kernel(node_features, Adj_block, node_prompt, parc_token, disease_emb, proj_W, proj_b, dis_W, dis_b, attn_Wqkv, attn_bqkv, attn_Wo, attn_bo, ln1_g, ln1_b, ln2_g, ln2_b, ffn_rW, ffn_rb, ffn_W1, ffn_b1, ffn_W2, ffn_b2, gcn_rW, gcn_rb, gcn_W, bn_g, bn_b, pred_W, pred_b):
    raise NotImplementedError("write your pallas kernel here")


# speedup vs baseline: 1.1576x; 1.1576x over previous
import jax
import jax.numpy as jnp
from jax.experimental import pallas as pl
from jax.experimental.pallas import tpu as pltpu

B = 64
N = 200
F = 200
H = 256
MAXF = 256
RW = 5
NH = 8
E = 8
G = 2
LP = 2
DFF = 1024
NC = 2
S = N + 2
SP = 208
HD = H // NH
NEG = -1e30


def _row_iota(shape, dim):
    return jax.lax.broadcasted_iota(jnp.int32, shape, dim)


def _ln(t, g, b):
    mu = jnp.mean(t, axis=-1, keepdims=True)
    var = jnp.mean((t - mu) ** 2, axis=-1, keepdims=True)
    return (t - mu) * jax.lax.rsqrt(var + 1e-5) * g + b


def _top1(scores):
    mx = jnp.max(scores, axis=-1, keepdims=True)
    ids = _row_iota(scores.shape, 1)
    cand = jnp.where(scores >= mx, ids, jnp.int32(E))
    return jnp.min(cand).astype(jnp.int32)


def _dot(a, b):
    return jax.lax.dot_general(a.astype(jnp.bfloat16), b.astype(jnp.bfloat16),
                               (((a.ndim - 1,), (0,)), ((), ())),
                               preferred_element_type=jnp.float32)


def _seq_mean(t):
    rows = _row_iota((SP, 1), 0)
    tm = jnp.where(rows < S, t, 0.0)
    return jnp.sum(tm, axis=0, keepdims=True) * (1.0 / S)


def _stage1_body(adj_ref, nf_ref, prompt_ref, projW_ref, projb_ref,
                 parc_ref, dis_ref, disW_ref, disb_ref, x_ref):
    a = adj_ref[0]
    adj = a / (jnp.sum(a, axis=-1, keepdims=True) + 1e-6)
    rows = _row_iota((SP, SP), 0)
    cols = _row_iota((SP, SP), 1)
    eye = (rows == cols).astype(jnp.float32)
    rw = adj
    diags = []
    for k in range(RW):
        diags.append(jnp.sum(rw * eye, axis=-1, keepdims=True))
        if k < RW - 1:
            rw = _dot(rw, adj)
    nf = nf_ref[0]
    colf = _row_iota((SP, MAXF), 1)
    for k in range(RW):
        nf = nf + jnp.where(colf == (F + k), diags[k], 0.0)
    nf = nf * prompt_ref[0]
    xp = _dot(nf, projW_ref[...]) + projb_ref[...]
    rid = _row_iota((SP, 1), 0)
    node_row = jnp.logical_and(rid >= 2, rid < S)
    xp = jnp.where(node_row, xp, 0.0)
    dis = _dot(dis_ref[...], disW_ref[...]) + disb_ref[...]
    parc = _dot(parc_ref[...], projW_ref[...]) + projb_ref[...]
    xp = xp + jnp.where(rid == 0, dis, 0.0) + jnp.where(rid == 1, parc, 0.0)
    x_ref[0] = xp


def _stage1(adj_shift, nf_pad, prompt, projW, projb, parc, dis, disW, disb):
    return pl.pallas_call(
        _stage1_body,
        grid=(B,),
        in_specs=[
            pl.BlockSpec((1, SP, SP), lambda b: (b, 0, 0)),
            pl.BlockSpec((1, SP, MAXF), lambda b: (b, 0, 0)),
            pl.BlockSpec((1, SP, MAXF), lambda b: (0, 0, 0)),
            pl.BlockSpec((MAXF, H), lambda b: (0, 0)),
            pl.BlockSpec((1, H), lambda b: (0, 0)),
            pl.BlockSpec((1, MAXF), lambda b: (0, 0)),
            pl.BlockSpec((1, 768), lambda b: (0, 0)),
            pl.BlockSpec((768, H), lambda b: (0, 0)),
            pl.BlockSpec((1, H), lambda b: (0, 0)),
        ],
        out_specs=pl.BlockSpec((1, SP, H), lambda b: (b, 0, 0)),
        out_shape=jax.ShapeDtypeStruct((B, SP, H), jnp.float32),
    )(adj_shift, nf_pad, prompt, projW, projb, parc, dis, disW, disb)


def _layer_body(x_ref, Wqkv_ref, bqkv_ref, Wo_ref, bo_ref, g1_ref, b1_ref,
                g2_ref, b2_ref, rW_ref, rb_ref, W1_ref, fb1_ref, W2_ref,
                fb2_ref, out_ref):
    x = x_ref[0]
    qkv = _dot(x, Wqkv_ref[...]) + bqkv_ref[...]
    cols = _row_iota((SP, SP), 1)
    key_mask = cols < S
    outs = []
    scale = 1.0 / (HD ** 0.5)
    for h in range(NH):
        qh = qkv[:, h * HD:(h + 1) * HD]
        kh = qkv[:, H + h * HD:H + (h + 1) * HD]
        vh = qkv[:, 2 * H + h * HD:2 * H + (h + 1) * HD]
        s = jax.lax.dot_general(qh.astype(jnp.bfloat16),
                                kh.astype(jnp.bfloat16),
                                (((1,), (1,)), ((), ())),
                                preferred_element_type=jnp.float32) * scale
        s = jnp.where(key_mask, s, NEG)
        m = jnp.max(s, axis=-1, keepdims=True)
        e = jnp.exp(s - m)
        p = e / jnp.sum(e, axis=-1, keepdims=True)
        outs.append(_dot(p, vh))
    o = jnp.concatenate(outs, axis=-1)
    a = _dot(o, Wo_ref[...]) + bo_ref[...]
    x = _ln(x + a, g1_ref[...], b1_ref[...])
    scores = _dot(_seq_mean(x), rW_ref[...]) + rb_ref[...]
    t1 = _top1(scores)
    w1 = W1_ref[t1]
    b1v = fb1_ref[pl.ds(t1, 1), :]
    w2 = W2_ref[t1]
    b2v = fb2_ref[pl.ds(t1, 1), :]
    hdn = jnp.maximum(_dot(x, w1) + b1v, 0.0)
    y = _dot(hdn, w2) + b2v
    x = _ln(x + y, g2_ref[...], b2_ref[...])
    rid = _row_iota((SP, 1), 0)
    out_ref[0] = jnp.where(rid < S, x, 0.0)


def _layer(x, Wqkv, bqkv, Wo, bo, g1, b1, g2, b2, rW, rb, W1, fb1, W2, fb2):
    c = lambda shape: pl.BlockSpec(shape, lambda b: (0,) * len(shape))
    return pl.pallas_call(
        _layer_body,
        grid=(B,),
        in_specs=[
            pl.BlockSpec((1, SP, H), lambda b: (b, 0, 0)),
            c((H, 3 * H)), c((1, 3 * H)), c((H, H)), c((1, H)),
            c((1, H)), c((1, H)), c((1, H)), c((1, H)),
            c((H, E)), c((1, E)),
            c((E, H, DFF)), c((E, DFF)), c((E, DFF, H)), c((E, H)),
        ],
        out_specs=pl.BlockSpec((1, SP, H), lambda b: (b, 0, 0)),
        out_shape=jax.ShapeDtypeStruct((B, SP, H), jnp.float32),
    )(x, Wqkv, bqkv, Wo, bo, g1, b1, g2, b2, rW, rb, W1, fb1, W2, fb2)


def _gcnA_body(x_ref, A_ref, W_ref, rW_ref, rb_ref,
               o_ref, sums_ref, sumsq_ref, cnt_ref):
    b = pl.program_id(0)

    @pl.when(b == 0)
    def _init():
        sums_ref[...] = jnp.zeros((E, H), jnp.float32)
        sumsq_ref[...] = jnp.zeros((E, H), jnp.float32)
        cnt_ref[...] = jnp.zeros((E, H), jnp.float32)

    x = x_ref[0]
    scores = _dot(_seq_mean(x), rW_ref[...]) + rb_ref[...]
    t1 = _top1(scores)
    w = W_ref[t1]
    o = _dot(A_ref[0], _dot(x, w))
    o_ref[0] = o
    rid = _row_iota((SP, 1), 0)
    om = jnp.where(rid < S, o, 0.0)
    srow = jnp.sum(om, axis=0, keepdims=True)
    sqrow = jnp.sum(om * om, axis=0, keepdims=True)
    onehot = (_row_iota((E, H), 0) == t1).astype(jnp.float32)
    sums_ref[...] += onehot * srow
    sumsq_ref[...] += onehot * sqrow
    cnt_ref[...] += onehot


def _gcnB_body(x_ref, o_ref, rW_ref, rb_ref, sums_ref, sumsq_ref, cnt_ref,
               g_ref, bb_ref, predW_ref, predb_ref, prev_ref,
               xout_ref, tot_ref):
    x = x_ref[0]
    scores = _dot(_seq_mean(x), rW_ref[...]) + rb_ref[...]
    t1 = _top1(scores)
    cnt = cnt_ref[pl.ds(t1, 1), :]
    cntS = jnp.maximum(cnt * jnp.float32(S), 1.0)
    mu = sums_ref[pl.ds(t1, 1), :] / cntS
    ex2 = sumsq_ref[pl.ds(t1, 1), :] / cntS
    var = ex2 - mu * mu
    o = o_ref[0]
    obn = (o - mu) * jax.lax.rsqrt(var + 1e-5) * g_ref[pl.ds(t1, 1), :] \
        + bb_ref[pl.ds(t1, 1), :]
    xn = jnp.maximum(obn, 0.0)
    rid = _row_iota((SP, 1), 0)
    xn = jnp.where(rid < S, xn, 0.0)
    xout_ref[0] = xn
    tot = prev_ref[0] + _dot(_seq_mean(xn), predW_ref[...]) + predb_ref[...]
    tot_ref[0] = tot


def _gcn(x, A, W, rW, rb, g, bb, predW, predb, prev):
    c = lambda shape: pl.BlockSpec(shape, lambda b: (0,) * len(shape))
    o, sums, sumsq, cnt = pl.pallas_call(
        _gcnA_body,
        grid=(B,),
        in_specs=[
            pl.BlockSpec((1, SP, H), lambda b: (b, 0, 0)),
            pl.BlockSpec((1, SP, SP), lambda b: (b, 0, 0)),
            c((E, H, H)), c((H, E)), c((1, E)),
        ],
        out_specs=[
            pl.BlockSpec((1, SP, H), lambda b: (b, 0, 0)),
            c((E, H)), c((E, H)), c((E, H)),
        ],
        out_shape=[
            jax.ShapeDtypeStruct((B, SP, H), jnp.float32),
            jax.ShapeDtypeStruct((E, H), jnp.float32),
            jax.ShapeDtypeStruct((E, H), jnp.float32),
            jax.ShapeDtypeStruct((E, H), jnp.float32),
        ],
        compiler_params=pltpu.CompilerParams(
            dimension_semantics=("arbitrary",)),
    )(x, A, W, rW, rb)
    xn, tot = pl.pallas_call(
        _gcnB_body,
        grid=(B,),
        in_specs=[
            pl.BlockSpec((1, SP, H), lambda b: (b, 0, 0)),
            pl.BlockSpec((1, SP, H), lambda b: (b, 0, 0)),
            c((H, E)), c((1, E)),
            c((E, H)), c((E, H)), c((E, H)),
            c((E, H)), c((E, H)),
            c((H, 128)), c((1, 128)),
            pl.BlockSpec((1, 1, 128), lambda b: (b, 0, 0)),
        ],
        out_specs=[
            pl.BlockSpec((1, SP, H), lambda b: (b, 0, 0)),
            pl.BlockSpec((1, 1, 128), lambda b: (b, 0, 0)),
        ],
        out_shape=[
            jax.ShapeDtypeStruct((B, SP, H), jnp.float32),
            jax.ShapeDtypeStruct((B, 1, 128), jnp.float32),
        ],
    )(x, o, rW, rb, sums, sumsq, cnt, g, bb, predW, predb, prev)
    return xn, tot


def kernel(node_features, Adj_block, node_prompt, parc_token, disease_emb,
           proj_W, proj_b, dis_W, dis_b, attn_Wqkv, attn_bqkv, attn_Wo,
           attn_bo, ln1_g, ln1_b, ln2_g, ln2_b, ffn_rW, ffn_rb, ffn_W1,
           ffn_b1, ffn_W2, ffn_b2, gcn_rW, gcn_rb, gcn_W, bn_g, bn_b,
           pred_W, pred_b):
    f32 = jnp.float32
    nf_pad = jnp.zeros((B, SP, MAXF), f32).at[:, 2:2 + N, :F].set(node_features)
    adj_shift = jnp.zeros((B, SP, SP), f32).at[:, 2:2 + N, 2:2 + N].set(Adj_block)
    adj_gcn = adj_shift.at[:, :2, :S].set(1.0).at[:, :S, :2].set(1.0)
    prompt = jnp.zeros((1, SP, MAXF), f32).at[0, 2:2 + N, :].set(node_prompt[0, :N, :])
    predW_pad = jnp.zeros((G, H, 128), f32).at[:, :, :NC].set(pred_W)
    predb_pad = jnp.zeros((G, 1, 128), f32).at[:, 0, :NC].set(pred_b)

    x = _stage1(adj_shift, nf_pad, prompt, proj_W,
                proj_b.reshape(1, H), parc_token.reshape(1, MAXF),
                disease_emb.reshape(1, 768), dis_W, dis_b.reshape(1, H))

    tot = jnp.zeros((B, 1, 128), f32)
    for gl in range(G):
        for al in range(LP):
            l = gl * LP + al
            x = _layer(x, attn_Wqkv[l], attn_bqkv[l].reshape(1, 3 * H),
                       attn_Wo[l], attn_bo[l].reshape(1, H),
                       ln1_g[l].reshape(1, H), ln1_b[l].reshape(1, H),
                       ln2_g[l].reshape(1, H), ln2_b[l].reshape(1, H),
                       ffn_rW[l], ffn_rb[l].reshape(1, E),
                       ffn_W1[l], ffn_b1[l], ffn_W2[l], ffn_b2[l])
        x, tot = _gcn(x, adj_gcn, gcn_W[gl], gcn_rW[gl],
                      gcn_rb[gl].reshape(1, E), bn_g[gl], bn_b[gl],
                      predW_pad[gl], predb_pad[gl], tot)
    return tot[:, 0, :NC]


# speedup vs baseline: 1.2697x; 1.0968x over previous
import jax
import jax.numpy as jnp
from jax.experimental import pallas as pl
from jax.experimental.pallas import tpu as pltpu

B = 64
N = 200
F = 200
H = 256
MAXF = 256
RW = 5
NH = 8
E = 8
G = 2
LP = 2
DFF = 1024
NC = 2
S = N + 2
SP = 208
HD = H // NH
NEG = -1e30
GPB = 2


def _row_iota(shape, dim):
    return jax.lax.broadcasted_iota(jnp.int32, shape, dim)


def _ln(t, g, b):
    mu = jnp.mean(t, axis=-1, keepdims=True)
    var = jnp.mean((t - mu) ** 2, axis=-1, keepdims=True)
    return (t - mu) * jax.lax.rsqrt(var + 1e-5) * g + b


def _top1(scores):
    mx = jnp.max(scores, axis=-1, keepdims=True)
    ids = _row_iota(scores.shape, 1)
    cand = jnp.where(scores >= mx, ids, jnp.int32(E))
    return jnp.min(cand).astype(jnp.int32)


def _dot(a, b):
    return jax.lax.dot_general(a.astype(jnp.bfloat16), b.astype(jnp.bfloat16),
                               (((a.ndim - 1,), (0,)), ((), ())),
                               preferred_element_type=jnp.float32)


def _dot_t(a, b):
    return jax.lax.dot_general(a.astype(jnp.bfloat16), b.astype(jnp.bfloat16),
                               (((1,), (1,)), ((), ())),
                               preferred_element_type=jnp.float32)


def _seq_mean(t):
    rows = _row_iota((SP, 1), 0)
    tm = jnp.where(rows < S, t, 0.0)
    return jnp.sum(tm, axis=0, keepdims=True) * (1.0 / S)


def _stage1_body(adj_ref, nf_ref, prompt_ref, projW_ref, projb_ref,
                 parc_ref, dis_ref, disW_ref, disb_ref, x_ref):
    rows = _row_iota((SP, SP), 0)
    cols = _row_iota((SP, SP), 1)
    eye = (rows == cols).astype(jnp.float32)
    rid = _row_iota((SP, 1), 0)
    node_row = jnp.logical_and(rid >= 2, rid < S)
    colf = _row_iota((SP, MAXF), 1)
    dis = _dot(dis_ref[...], disW_ref[...]) + disb_ref[...]
    parc = _dot(parc_ref[...], projW_ref[...]) + projb_ref[...]
    for i in range(GPB):
        a = adj_ref[i]
        adj = a / (jnp.sum(a, axis=-1, keepdims=True) + 1e-6)
        rw = adj
        diags = []
        for k in range(RW):
            diags.append(jnp.sum(rw * eye, axis=-1, keepdims=True))
            if k < RW - 1:
                rw = _dot(rw, adj)
        nf = nf_ref[i]
        for k in range(RW):
            nf = nf + jnp.where(colf == (F + k), diags[k], 0.0)
        nf = nf * prompt_ref[0]
        xp = _dot(nf, projW_ref[...]) + projb_ref[...]
        xp = jnp.where(node_row, xp, 0.0)
        xp = xp + jnp.where(rid == 0, dis, 0.0) + jnp.where(rid == 1, parc, 0.0)
        x_ref[i] = xp


def _stage1(adj_shift, nf_pad, prompt, projW, projb, parc, dis, disW, disb):
    return pl.pallas_call(
        _stage1_body,
        grid=(B // GPB,),
        in_specs=[
            pl.BlockSpec((GPB, SP, SP), lambda b: (b, 0, 0)),
            pl.BlockSpec((GPB, SP, MAXF), lambda b: (b, 0, 0)),
            pl.BlockSpec((1, SP, MAXF), lambda b: (0, 0, 0)),
            pl.BlockSpec((MAXF, H), lambda b: (0, 0)),
            pl.BlockSpec((1, H), lambda b: (0, 0)),
            pl.BlockSpec((1, MAXF), lambda b: (0, 0)),
            pl.BlockSpec((1, 768), lambda b: (0, 0)),
            pl.BlockSpec((768, H), lambda b: (0, 0)),
            pl.BlockSpec((1, H), lambda b: (0, 0)),
        ],
        out_specs=pl.BlockSpec((GPB, SP, H), lambda b: (b, 0, 0)),
        out_shape=jax.ShapeDtypeStruct((B, SP, H), jnp.float32),
    )(adj_shift, nf_pad, prompt, projW, projb, parc, dis, disW, disb)


def _layer_body(x_ref, Wqkv_ref, bqkv_ref, Wo_ref, bo_ref, g1_ref, b1_ref,
                g2_ref, b2_ref, rW_ref, rb_ref, W1_ref, fb1_ref, W2_ref,
                fb2_ref, out_ref):
    cols = _row_iota((SP, SP), 1)
    key_mask = cols < S
    rid = _row_iota((SP, 1), 0)
    scale = 1.0 / (HD ** 0.5)
    for i in range(GPB):
        x = x_ref[i]
        qkv = _dot(x, Wqkv_ref[...]) + bqkv_ref[...]
        outs = []
        for h in range(NH):
            qh = qkv[:, h * HD:(h + 1) * HD]
            kh = qkv[:, H + h * HD:H + (h + 1) * HD]
            vh = qkv[:, 2 * H + h * HD:2 * H + (h + 1) * HD]
            s = _dot_t(qh, kh) * scale
            s = jnp.where(key_mask, s, NEG)
            m = jnp.max(s, axis=-1, keepdims=True)
            e = jnp.exp(s - m)
            p = e / jnp.sum(e, axis=-1, keepdims=True)
            outs.append(_dot(p, vh))
        o = jnp.concatenate(outs, axis=-1)
        a = _dot(o, Wo_ref[...]) + bo_ref[...]
        x = _ln(x + a, g1_ref[...], b1_ref[...])
        scores = _dot(_seq_mean(x), rW_ref[...]) + rb_ref[...]
        t1 = _top1(scores)
        w1 = W1_ref[t1]
        b1v = fb1_ref[pl.ds(t1, 1), :]
        w2 = W2_ref[t1]
        b2v = fb2_ref[pl.ds(t1, 1), :]
        hdn = jnp.maximum(_dot(x, w1) + b1v, 0.0)
        y = _dot(hdn, w2) + b2v
        x = _ln(x + y, g2_ref[...], b2_ref[...])
        out_ref[i] = jnp.where(rid < S, x, 0.0)


def _layer(x, Wqkv, bqkv, Wo, bo, g1, b1, g2, b2, rW, rb, W1, fb1, W2, fb2):
    c = lambda shape: pl.BlockSpec(shape, lambda b: (0,) * len(shape))
    return pl.pallas_call(
        _layer_body,
        grid=(B // GPB,),
        in_specs=[
            pl.BlockSpec((GPB, SP, H), lambda b: (b, 0, 0)),
            c((H, 3 * H)), c((1, 3 * H)), c((H, H)), c((1, H)),
            c((1, H)), c((1, H)), c((1, H)), c((1, H)),
            c((H, E)), c((1, E)),
            c((E, H, DFF)), c((E, DFF)), c((E, DFF, H)), c((E, H)),
        ],
        out_specs=pl.BlockSpec((GPB, SP, H), lambda b: (b, 0, 0)),
        out_shape=jax.ShapeDtypeStruct((B, SP, H), jnp.float32),
    )(x, Wqkv, bqkv, Wo, bo, g1, b1, g2, b2, rW, rb, W1, fb1, W2, fb2)


def _gcnA_body(x_ref, A_ref, W_ref, rW_ref, rb_ref,
               o_ref, sums_ref, sumsq_ref, cnt_ref):
    b = pl.program_id(0)

    @pl.when(b == 0)
    def _init():
        sums_ref[...] = jnp.zeros((E, H), jnp.float32)
        sumsq_ref[...] = jnp.zeros((E, H), jnp.float32)
        cnt_ref[...] = jnp.zeros((E, H), jnp.float32)

    rid = _row_iota((SP, 1), 0)
    for i in range(GPB):
        x = x_ref[i]
        scores = _dot(_seq_mean(x), rW_ref[...]) + rb_ref[...]
        t1 = _top1(scores)
        w = W_ref[t1]
        o = _dot(A_ref[i], _dot(x, w))
        o_ref[i] = o
        om = jnp.where(rid < S, o, 0.0)
        srow = jnp.sum(om, axis=0, keepdims=True)
        sqrow = jnp.sum(om * om, axis=0, keepdims=True)
        onehot = (_row_iota((E, H), 0) == t1).astype(jnp.float32)
        sums_ref[...] += onehot * srow
        sumsq_ref[...] += onehot * sqrow
        cnt_ref[...] += onehot


def _gcnB_body(x_ref, o_ref, rW_ref, rb_ref, sums_ref, sumsq_ref, cnt_ref,
               g_ref, bb_ref, predW_ref, predb_ref, prev_ref,
               xout_ref, tot_ref):
    rid = _row_iota((SP, 1), 0)
    for i in range(GPB):
        x = x_ref[i]
        scores = _dot(_seq_mean(x), rW_ref[...]) + rb_ref[...]
        t1 = _top1(scores)
        cnt = cnt_ref[pl.ds(t1, 1), :]
        cntS = jnp.maximum(cnt * jnp.float32(S), 1.0)
        mu = sums_ref[pl.ds(t1, 1), :] / cntS
        ex2 = sumsq_ref[pl.ds(t1, 1), :] / cntS
        var = ex2 - mu * mu
        o = o_ref[i]
        obn = (o - mu) * jax.lax.rsqrt(var + 1e-5) * g_ref[pl.ds(t1, 1), :] \
            + bb_ref[pl.ds(t1, 1), :]
        xn = jnp.maximum(obn, 0.0)
        xn = jnp.where(rid < S, xn, 0.0)
        xout_ref[i] = xn
        tot = prev_ref[i] + _dot(_seq_mean(xn), predW_ref[...]) + predb_ref[...]
        tot_ref[i] = tot


def _gcn(x, A, W, rW, rb, g, bb, predW, predb, prev):
    c = lambda shape: pl.BlockSpec(shape, lambda b: (0,) * len(shape))
    o, sums, sumsq, cnt = pl.pallas_call(
        _gcnA_body,
        grid=(B // GPB,),
        in_specs=[
            pl.BlockSpec((GPB, SP, H), lambda b: (b, 0, 0)),
            pl.BlockSpec((GPB, SP, SP), lambda b: (b, 0, 0)),
            c((E, H, H)), c((H, E)), c((1, E)),
        ],
        out_specs=[
            pl.BlockSpec((GPB, SP, H), lambda b: (b, 0, 0)),
            c((E, H)), c((E, H)), c((E, H)),
        ],
        out_shape=[
            jax.ShapeDtypeStruct((B, SP, H), jnp.float32),
            jax.ShapeDtypeStruct((E, H), jnp.float32),
            jax.ShapeDtypeStruct((E, H), jnp.float32),
            jax.ShapeDtypeStruct((E, H), jnp.float32),
        ],
        compiler_params=pltpu.CompilerParams(
            dimension_semantics=("arbitrary",)),
    )(x, A, W, rW, rb)
    xn, tot = pl.pallas_call(
        _gcnB_body,
        grid=(B // GPB,),
        in_specs=[
            pl.BlockSpec((GPB, SP, H), lambda b: (b, 0, 0)),
            pl.BlockSpec((GPB, SP, H), lambda b: (b, 0, 0)),
            c((H, E)), c((1, E)),
            c((E, H)), c((E, H)), c((E, H)),
            c((E, H)), c((E, H)),
            c((H, 128)), c((1, 128)),
            pl.BlockSpec((GPB, 1, 128), lambda b: (b, 0, 0)),
        ],
        out_specs=[
            pl.BlockSpec((GPB, SP, H), lambda b: (b, 0, 0)),
            pl.BlockSpec((GPB, 1, 128), lambda b: (b, 0, 0)),
        ],
        out_shape=[
            jax.ShapeDtypeStruct((B, SP, H), jnp.float32),
            jax.ShapeDtypeStruct((B, 1, 128), jnp.float32),
        ],
    )(x, o, rW, rb, sums, sumsq, cnt, g, bb, predW, predb, prev)
    return xn, tot


def kernel(node_features, Adj_block, node_prompt, parc_token, disease_emb,
           proj_W, proj_b, dis_W, dis_b, attn_Wqkv, attn_bqkv, attn_Wo,
           attn_bo, ln1_g, ln1_b, ln2_g, ln2_b, ffn_rW, ffn_rb, ffn_W1,
           ffn_b1, ffn_W2, ffn_b2, gcn_rW, gcn_rb, gcn_W, bn_g, bn_b,
           pred_W, pred_b):
    f32 = jnp.float32
    nf_pad = jnp.zeros((B, SP, MAXF), f32).at[:, 2:2 + N, :F].set(node_features)
    adj_shift = jnp.zeros((B, SP, SP), f32).at[:, 2:2 + N, 2:2 + N].set(Adj_block)
    adj_gcn = adj_shift.at[:, :2, :S].set(1.0).at[:, :S, :2].set(1.0)
    prompt = jnp.zeros((1, SP, MAXF), f32).at[0, 2:2 + N, :].set(node_prompt[0, :N, :])
    predW_pad = jnp.zeros((G, H, 128), f32).at[:, :, :NC].set(pred_W)
    predb_pad = jnp.zeros((G, 1, 128), f32).at[:, 0, :NC].set(pred_b)

    x = _stage1(adj_shift, nf_pad, prompt, proj_W,
                proj_b.reshape(1, H), parc_token.reshape(1, MAXF),
                disease_emb.reshape(1, 768), dis_W, dis_b.reshape(1, H))

    tot = jnp.zeros((B, 1, 128), f32)
    for gl in range(G):
        for al in range(LP):
            l = gl * LP + al
            x = _layer(x, attn_Wqkv[l], attn_bqkv[l].reshape(1, 3 * H),
                       attn_Wo[l], attn_bo[l].reshape(1, H),
                       ln1_g[l].reshape(1, H), ln1_b[l].reshape(1, H),
                       ln2_g[l].reshape(1, H), ln2_b[l].reshape(1, H),
                       ffn_rW[l], ffn_rb[l].reshape(1, E),
                       ffn_W1[l], ffn_b1[l], ffn_W2[l], ffn_b2[l])
        x, tot = _gcn(x, adj_gcn, gcn_W[gl], gcn_rW[gl],
                      gcn_rb[gl].reshape(1, E), bn_g[gl], bn_b[gl],
                      predW_pad[gl], predb_pad[gl], tot)
    return tot[:, 0, :NC]


# speedup vs baseline: 1.8238x; 1.4365x over previous
import jax
import jax.numpy as jnp
from jax.experimental import pallas as pl
from jax.experimental.pallas import tpu as pltpu

B = 64
N = 200
F = 200
H = 256
MAXF = 256
RW = 5
NH = 8
E = 8
G = 2
LP = 2
DFF = 1024
NC = 2
S = N + 2
SP = 208
HD = H // NH
NEG = -1e30
GPB = 2


def _row_iota(shape, dim):
    return jax.lax.broadcasted_iota(jnp.int32, shape, dim)


def _ln(t, g, b):
    mu = jnp.mean(t, axis=-1, keepdims=True)
    var = jnp.mean((t - mu) ** 2, axis=-1, keepdims=True)
    return (t - mu) * jax.lax.rsqrt(var + 1e-5) * g + b


def _top1(scores):
    mx = jnp.max(scores, axis=-1, keepdims=True)
    ids = _row_iota(scores.shape, 1)
    cand = jnp.where(scores >= mx, ids, jnp.int32(E))
    return jnp.min(cand).astype(jnp.int32)


def _bf(a):
    return a if a.dtype == jnp.bfloat16 else a.astype(jnp.bfloat16)


def _dot(a, b):
    return jax.lax.dot_general(_bf(a), _bf(b),
                               (((a.ndim - 1,), (0,)), ((), ())),
                               preferred_element_type=jnp.float32)


def _dot_t(a, b):
    return jax.lax.dot_general(_bf(a), _bf(b),
                               (((1,), (1,)), ((), ())),
                               preferred_element_type=jnp.float32)


def _seq_mean(t):
    rows = _row_iota((SP, 1), 0)
    tm = jnp.where(rows < S, t, 0.0)
    return jnp.sum(tm, axis=0, keepdims=True) * (1.0 / S)


def _stage1_body(adj_ref, nf_ref, prompt_ref, projW_ref, projb_ref,
                 parc_ref, dis_ref, disW_ref, disb_ref, x_ref):
    rows = _row_iota((SP, SP), 0)
    cols = _row_iota((SP, SP), 1)
    eye = (rows == cols).astype(jnp.float32)
    rid = _row_iota((SP, 1), 0)
    node_row = jnp.logical_and(rid >= 2, rid < S)
    colf = _row_iota((SP, MAXF), 1)
    dis = _dot(dis_ref[...], disW_ref[...]) + disb_ref[...]
    parc = _dot(parc_ref[...], projW_ref[...]) + projb_ref[...]
    for i in range(GPB):
        a = adj_ref[i]
        adj = a / (jnp.sum(a, axis=-1, keepdims=True) + 1e-6)
        rw = adj
        diags = []
        for k in range(RW):
            diags.append(jnp.sum(rw * eye, axis=-1, keepdims=True))
            if k < RW - 1:
                rw = _dot(rw, adj)
        nf = nf_ref[i]
        for k in range(RW):
            nf = nf + jnp.where(colf == (F + k), diags[k], 0.0)
        nf = nf * prompt_ref[0]
        xp = _dot(nf, projW_ref[...]) + projb_ref[...]
        xp = jnp.where(node_row, xp, 0.0)
        xp = xp + jnp.where(rid == 0, dis, 0.0) + jnp.where(rid == 1, parc, 0.0)
        x_ref[i] = xp


def _stage1(adj_shift, nf_pad, prompt, projW, projb, parc, dis, disW, disb):
    return pl.pallas_call(
        _stage1_body,
        grid=(B // GPB,),
        in_specs=[
            pl.BlockSpec((GPB, SP, SP), lambda b: (b, 0, 0)),
            pl.BlockSpec((GPB, SP, MAXF), lambda b: (b, 0, 0)),
            pl.BlockSpec((1, SP, MAXF), lambda b: (0, 0, 0)),
            pl.BlockSpec((MAXF, H), lambda b: (0, 0)),
            pl.BlockSpec((1, H), lambda b: (0, 0)),
            pl.BlockSpec((1, MAXF), lambda b: (0, 0)),
            pl.BlockSpec((1, 768), lambda b: (0, 0)),
            pl.BlockSpec((768, H), lambda b: (0, 0)),
            pl.BlockSpec((1, H), lambda b: (0, 0)),
        ],
        out_specs=pl.BlockSpec((GPB, SP, H), lambda b: (b, 0, 0)),
        out_shape=jax.ShapeDtypeStruct((B, SP, H), jnp.float32),
    )(adj_shift, nf_pad, prompt, projW, projb, parc, dis, disW, disb)


def _layer_body(x_ref, Wqkv_ref, bqkv_ref, Wo_ref, bo_ref, g1_ref, b1_ref,
                g2_ref, b2_ref, rW_ref, rb_ref, W1_ref, fb1_ref, W2_ref,
                fb2_ref, out_ref):
    cols = _row_iota((SP, SP), 1)
    key_maskf = (cols < S).astype(jnp.float32)
    rid = _row_iota((SP, 1), 0)
    scale = 1.0 / (HD ** 0.5)
    ones_col = jnp.ones((SP, 1), jnp.float32)
    for i in range(GPB):
        x = x_ref[i]
        qkv = _dot(x, Wqkv_ref[...]) + bqkv_ref[...]
        outs = []
        for h in range(NH):
            qh = qkv[:, h * HD:(h + 1) * HD]
            kh = qkv[:, H + h * HD:H + (h + 1) * HD]
            vh = qkv[:, 2 * H + h * HD:2 * H + (h + 1) * HD]
            s = _dot_t(qh, kh) * scale
            e = jnp.exp(s) * key_maskf
            oe = _dot(e, jnp.concatenate([vh, ones_col], axis=-1))
            outs.append(oe[:, :HD] / oe[:, HD:HD + 1])
        o = jnp.concatenate(outs, axis=-1)
        a = _dot(o, Wo_ref[...]) + bo_ref[...]
        x = _ln(x + a, g1_ref[...], b1_ref[...])
        scores = _dot(_seq_mean(x), rW_ref[...]) + rb_ref[...]
        t1 = _top1(scores)
        w1 = W1_ref[t1]
        b1v = fb1_ref[pl.ds(t1, 1), :]
        w2 = W2_ref[t1]
        b2v = fb2_ref[pl.ds(t1, 1), :]
        hdn = jnp.maximum(_dot(x, w1) + b1v, 0.0)
        y = _dot(hdn, w2) + b2v
        x = _ln(x + y, g2_ref[...], b2_ref[...])
        out_ref[i] = jnp.where(rid < S, x, 0.0)


def _layer(x, Wqkv, bqkv, Wo, bo, g1, b1, g2, b2, rW, rb, W1, fb1, W2, fb2):
    c = lambda shape: pl.BlockSpec(shape, lambda b: (0,) * len(shape))
    return pl.pallas_call(
        _layer_body,
        grid=(B // GPB,),
        in_specs=[
            pl.BlockSpec((GPB, SP, H), lambda b: (b, 0, 0)),
            c((H, 3 * H)), c((1, 3 * H)), c((H, H)), c((1, H)),
            c((1, H)), c((1, H)), c((1, H)), c((1, H)),
            c((H, E)), c((1, E)),
            c((E, H, DFF)), c((E, DFF)), c((E, DFF, H)), c((E, H)),
        ],
        out_specs=pl.BlockSpec((GPB, SP, H), lambda b: (b, 0, 0)),
        out_shape=jax.ShapeDtypeStruct((B, SP, H), jnp.float32),
    )(x, Wqkv, bqkv, Wo, bo, g1, b1, g2, b2, rW, rb, W1, fb1, W2, fb2)


def _gcnA_body(x_ref, A_ref, W_ref, rW_ref, rb_ref,
               o_ref, sums_ref, sumsq_ref, cnt_ref):
    b = pl.program_id(0)

    @pl.when(b == 0)
    def _init():
        sums_ref[...] = jnp.zeros((E, H), jnp.float32)
        sumsq_ref[...] = jnp.zeros((E, H), jnp.float32)
        cnt_ref[...] = jnp.zeros((E, H), jnp.float32)

    rid = _row_iota((SP, 1), 0)
    for i in range(GPB):
        x = x_ref[i]
        scores = _dot(_seq_mean(x), rW_ref[...]) + rb_ref[...]
        t1 = _top1(scores)
        w = W_ref[t1]
        o = _dot(A_ref[i], _dot(x, w))
        o_ref[i] = o
        om = jnp.where(rid < S, o, 0.0)
        srow = jnp.sum(om, axis=0, keepdims=True)
        sqrow = jnp.sum(om * om, axis=0, keepdims=True)
        onehot = (_row_iota((E, H), 0) == t1).astype(jnp.float32)
        sums_ref[...] += onehot * srow
        sumsq_ref[...] += onehot * sqrow
        cnt_ref[...] += onehot


def _gcnB_body(x_ref, o_ref, rW_ref, rb_ref, sums_ref, sumsq_ref, cnt_ref,
               g_ref, bb_ref, predW_ref, predb_ref, prev_ref,
               xout_ref, tot_ref):
    rid = _row_iota((SP, 1), 0)
    for i in range(GPB):
        x = x_ref[i]
        scores = _dot(_seq_mean(x), rW_ref[...]) + rb_ref[...]
        t1 = _top1(scores)
        cnt = cnt_ref[pl.ds(t1, 1), :]
        cntS = jnp.maximum(cnt * jnp.float32(S), 1.0)
        mu = sums_ref[pl.ds(t1, 1), :] / cntS
        ex2 = sumsq_ref[pl.ds(t1, 1), :] / cntS
        var = ex2 - mu * mu
        o = o_ref[i]
        obn = (o - mu) * jax.lax.rsqrt(var + 1e-5) * g_ref[pl.ds(t1, 1), :] \
            + bb_ref[pl.ds(t1, 1), :]
        xn = jnp.maximum(obn, 0.0)
        xn = jnp.where(rid < S, xn, 0.0)
        xout_ref[i] = xn
        tot = prev_ref[i] + _dot(_seq_mean(xn), predW_ref[...]) + predb_ref[...]
        tot_ref[i] = tot


def _gcn(x, A, W, rW, rb, g, bb, predW, predb, prev):
    c = lambda shape: pl.BlockSpec(shape, lambda b: (0,) * len(shape))
    o, sums, sumsq, cnt = pl.pallas_call(
        _gcnA_body,
        grid=(B // GPB,),
        in_specs=[
            pl.BlockSpec((GPB, SP, H), lambda b: (b, 0, 0)),
            pl.BlockSpec((GPB, SP, SP), lambda b: (b, 0, 0)),
            c((E, H, H)), c((H, E)), c((1, E)),
        ],
        out_specs=[
            pl.BlockSpec((GPB, SP, H), lambda b: (b, 0, 0)),
            c((E, H)), c((E, H)), c((E, H)),
        ],
        out_shape=[
            jax.ShapeDtypeStruct((B, SP, H), jnp.float32),
            jax.ShapeDtypeStruct((E, H), jnp.float32),
            jax.ShapeDtypeStruct((E, H), jnp.float32),
            jax.ShapeDtypeStruct((E, H), jnp.float32),
        ],
        compiler_params=pltpu.CompilerParams(
            dimension_semantics=("arbitrary",)),
    )(x, A, W, rW, rb)
    xn, tot = pl.pallas_call(
        _gcnB_body,
        grid=(B // GPB,),
        in_specs=[
            pl.BlockSpec((GPB, SP, H), lambda b: (b, 0, 0)),
            pl.BlockSpec((GPB, SP, H), lambda b: (b, 0, 0)),
            c((H, E)), c((1, E)),
            c((E, H)), c((E, H)), c((E, H)),
            c((E, H)), c((E, H)),
            c((H, 128)), c((1, 128)),
            pl.BlockSpec((GPB, 1, 128), lambda b: (b, 0, 0)),
        ],
        out_specs=[
            pl.BlockSpec((GPB, SP, H), lambda b: (b, 0, 0)),
            pl.BlockSpec((GPB, 1, 128), lambda b: (b, 0, 0)),
        ],
        out_shape=[
            jax.ShapeDtypeStruct((B, SP, H), jnp.float32),
            jax.ShapeDtypeStruct((B, 1, 128), jnp.float32),
        ],
    )(x, o, rW, rb, sums, sumsq, cnt, g, bb, predW, predb, prev)
    return xn, tot


def kernel(node_features, Adj_block, node_prompt, parc_token, disease_emb,
           proj_W, proj_b, dis_W, dis_b, attn_Wqkv, attn_bqkv, attn_Wo,
           attn_bo, ln1_g, ln1_b, ln2_g, ln2_b, ffn_rW, ffn_rb, ffn_W1,
           ffn_b1, ffn_W2, ffn_b2, gcn_rW, gcn_rb, gcn_W, bn_g, bn_b,
           pred_W, pred_b):
    f32 = jnp.float32
    bf16 = jnp.bfloat16
    nf_pad = jnp.zeros((B, SP, MAXF), f32).at[:, 2:2 + N, :F].set(node_features)
    adj_shift = jnp.zeros((B, SP, SP), f32).at[:, 2:2 + N, 2:2 + N].set(Adj_block)
    adj_gcn = adj_shift.at[:, :2, :S].set(1.0).at[:, :S, :2].set(1.0).astype(bf16)
    prompt = jnp.zeros((1, SP, MAXF), f32).at[0, 2:2 + N, :].set(node_prompt[0, :N, :])
    predW_pad = jnp.zeros((G, H, 128), f32).at[:, :, :NC].set(pred_W).astype(bf16)
    predb_pad = jnp.zeros((G, 1, 128), f32).at[:, 0, :NC].set(pred_b)
    proj_W = proj_W.astype(bf16)
    dis_W = dis_W.astype(bf16)
    attn_Wqkv = attn_Wqkv.astype(bf16)
    attn_Wo = attn_Wo.astype(bf16)
    ffn_rW = ffn_rW.astype(bf16)
    ffn_W1 = ffn_W1.astype(bf16)
    ffn_W2 = ffn_W2.astype(bf16)
    gcn_rW = gcn_rW.astype(bf16)
    gcn_W = gcn_W.astype(bf16)

    x = _stage1(adj_shift, nf_pad, prompt, proj_W,
                proj_b.reshape(1, H), parc_token.reshape(1, MAXF),
                disease_emb.reshape(1, 768), dis_W, dis_b.reshape(1, H))

    tot = jnp.zeros((B, 1, 128), f32)
    for gl in range(G):
        for al in range(LP):
            l = gl * LP + al
            x = _layer(x, attn_Wqkv[l], attn_bqkv[l].reshape(1, 3 * H),
                       attn_Wo[l], attn_bo[l].reshape(1, H),
                       ln1_g[l].reshape(1, H), ln1_b[l].reshape(1, H),
                       ln2_g[l].reshape(1, H), ln2_b[l].reshape(1, H),
                       ffn_rW[l], ffn_rb[l].reshape(1, E),
                       ffn_W1[l], ffn_b1[l], ffn_W2[l], ffn_b2[l])
        x, tot = _gcn(x, adj_gcn, gcn_W[gl], gcn_rW[gl],
                      gcn_rb[gl].reshape(1, E), bn_g[gl], bn_b[gl],
                      predW_pad[gl], predb_pad[gl], tot)
    return tot[:, 0, :NC]


# speedup vs baseline: 1.9664x; 1.0782x over previous
import jax
import jax.numpy as jnp
from jax.experimental import pallas as pl
from jax.experimental.pallas import tpu as pltpu

B = 64
N = 200
F = 200
H = 256
MAXF = 256
RW = 5
NH = 8
E = 8
G = 2
LP = 2
DFF = 1024
NC = 2
S = N + 2
SP = 208
HD = H // NH
NEG = -1e30
GPB = 4


def _row_iota(shape, dim):
    return jax.lax.broadcasted_iota(jnp.int32, shape, dim)


def _ln(t, g, b):
    mu = jnp.mean(t, axis=-1, keepdims=True)
    var = jnp.mean((t - mu) ** 2, axis=-1, keepdims=True)
    return (t - mu) * jax.lax.rsqrt(var + 1e-5) * g + b


def _top1(scores):
    mx = jnp.max(scores, axis=-1, keepdims=True)
    ids = _row_iota(scores.shape, 1)
    cand = jnp.where(scores >= mx, ids, jnp.int32(E))
    return jnp.min(cand).astype(jnp.int32)


def _bf(a):
    return a if a.dtype == jnp.bfloat16 else a.astype(jnp.bfloat16)


def _dot(a, b):
    return jax.lax.dot_general(_bf(a), _bf(b),
                               (((a.ndim - 1,), (0,)), ((), ())),
                               preferred_element_type=jnp.float32)


def _dot_t(a, b):
    return jax.lax.dot_general(_bf(a), _bf(b),
                               (((1,), (1,)), ((), ())),
                               preferred_element_type=jnp.float32)


def _seq_mean(t):
    rows = _row_iota((SP, 1), 0)
    tm = jnp.where(rows < S, t, 0.0)
    return jnp.sum(tm, axis=0, keepdims=True) * (1.0 / S)


def _stage1_body(adj_ref, nf_ref, prompt_ref, projW_ref, projb_ref,
                 parc_ref, dis_ref, disW_ref, disb_ref, x_ref):
    rows = _row_iota((SP, SP), 0)
    cols = _row_iota((SP, SP), 1)
    eye = (rows == cols).astype(jnp.float32)
    rid = _row_iota((SP, 1), 0)
    node_row = jnp.logical_and(rid >= 2, rid < S)
    colf = _row_iota((SP, MAXF), 1)
    dis = _dot(dis_ref[...], disW_ref[...]) + disb_ref[...]
    parc = _dot(parc_ref[...], projW_ref[...]) + projb_ref[...]
    for i in range(GPB):
        a = adj_ref[i]
        adj = a / (jnp.sum(a, axis=-1, keepdims=True) + 1e-6)
        rw = adj
        diags = []
        for k in range(RW):
            diags.append(jnp.sum(rw * eye, axis=-1, keepdims=True))
            if k < RW - 1:
                rw = _dot(rw, adj)
        nf = nf_ref[i]
        for k in range(RW):
            nf = nf + jnp.where(colf == (F + k), diags[k], 0.0)
        nf = nf * prompt_ref[0]
        xp = _dot(nf, projW_ref[...]) + projb_ref[...]
        xp = jnp.where(node_row, xp, 0.0)
        xp = xp + jnp.where(rid == 0, dis, 0.0) + jnp.where(rid == 1, parc, 0.0)
        x_ref[i] = xp


def _stage1(adj_shift, nf_pad, prompt, projW, projb, parc, dis, disW, disb):
    return pl.pallas_call(
        _stage1_body,
        grid=(B // GPB,),
        in_specs=[
            pl.BlockSpec((GPB, SP, SP), lambda b: (b, 0, 0)),
            pl.BlockSpec((GPB, SP, MAXF), lambda b: (b, 0, 0)),
            pl.BlockSpec((1, SP, MAXF), lambda b: (0, 0, 0)),
            pl.BlockSpec((MAXF, H), lambda b: (0, 0)),
            pl.BlockSpec((1, H), lambda b: (0, 0)),
            pl.BlockSpec((1, MAXF), lambda b: (0, 0)),
            pl.BlockSpec((1, 768), lambda b: (0, 0)),
            pl.BlockSpec((768, H), lambda b: (0, 0)),
            pl.BlockSpec((1, H), lambda b: (0, 0)),
        ],
        out_specs=pl.BlockSpec((GPB, SP, H), lambda b: (b, 0, 0)),
        out_shape=jax.ShapeDtypeStruct((B, SP, H), jnp.float32),
    )(adj_shift, nf_pad, prompt, projW, projb, parc, dis, disW, disb)


def _layer_body(x_ref, Wqkv_ref, bqkv_ref, Wo_ref, bo_ref, g1_ref, b1_ref,
                g2_ref, b2_ref, rW_ref, rb_ref, W1_ref, fb1_ref, W2_ref,
                fb2_ref, out_ref):
    rid = _row_iota((SP, 1), 0)
    scale = 1.0 / (HD ** 0.5)
    ones_col = jnp.ones((SP, 1), jnp.float32)
    for i in range(GPB):
        x = x_ref[i]
        qkv = _dot(x, Wqkv_ref[...]) + bqkv_ref[...]
        outs = []
        for h in range(NH):
            qh = qkv[:, h * HD:(h + 1) * HD] * scale
            kh = qkv[:, H + h * HD:H + (h + 1) * HD]
            vh = qkv[:, 2 * H + h * HD:2 * H + (h + 1) * HD]
            s = _dot_t(qh, kh)
            e = jnp.exp(s)
            oe = _dot(e, jnp.concatenate([vh, ones_col], axis=-1))
            outs.append(oe[:, :HD] / (oe[:, HD:HD + 1] - (SP - S)))
        o = jnp.concatenate(outs, axis=-1)
        a = _dot(o, Wo_ref[...]) + bo_ref[...]
        x = _ln(x + a, g1_ref[...], b1_ref[...])
        scores = _dot(_seq_mean(x), rW_ref[...]) + rb_ref[...]
        t1 = _top1(scores)
        w1 = W1_ref[t1]
        b1v = fb1_ref[pl.ds(t1, 1), :]
        w2 = W2_ref[t1]
        b2v = fb2_ref[pl.ds(t1, 1), :]
        hdn = jnp.maximum(_dot(x, w1) + b1v, 0.0)
        y = _dot(hdn, w2) + b2v
        x = _ln(x + y, g2_ref[...], b2_ref[...])
        out_ref[i] = jnp.where(rid < S, x, 0.0)


def _layer(x, Wqkv, bqkv, Wo, bo, g1, b1, g2, b2, rW, rb, W1, fb1, W2, fb2):
    c = lambda shape: pl.BlockSpec(shape, lambda b: (0,) * len(shape))
    return pl.pallas_call(
        _layer_body,
        grid=(B // GPB,),
        in_specs=[
            pl.BlockSpec((GPB, SP, H), lambda b: (b, 0, 0)),
            c((H, 3 * H)), c((1, 3 * H)), c((H, H)), c((1, H)),
            c((1, H)), c((1, H)), c((1, H)), c((1, H)),
            c((H, E)), c((1, E)),
            c((E, H, DFF)), c((E, DFF)), c((E, DFF, H)), c((E, H)),
        ],
        out_specs=pl.BlockSpec((GPB, SP, H), lambda b: (b, 0, 0)),
        out_shape=jax.ShapeDtypeStruct((B, SP, H), jnp.float32),
    )(x, Wqkv, bqkv, Wo, bo, g1, b1, g2, b2, rW, rb, W1, fb1, W2, fb2)


def _gcnA_body(x_ref, A_ref, W_ref, rW_ref, rb_ref,
               o_ref, sums_ref, sumsq_ref, cnt_ref):
    b = pl.program_id(0)

    @pl.when(b == 0)
    def _init():
        sums_ref[...] = jnp.zeros((E, H), jnp.float32)
        sumsq_ref[...] = jnp.zeros((E, H), jnp.float32)
        cnt_ref[...] = jnp.zeros((E, H), jnp.float32)

    rid = _row_iota((SP, 1), 0)
    for i in range(GPB):
        x = x_ref[i]
        scores = _dot(_seq_mean(x), rW_ref[...]) + rb_ref[...]
        t1 = _top1(scores)
        w = W_ref[t1]
        o = _dot(A_ref[i], _dot(x, w))
        o_ref[i] = o
        om = jnp.where(rid < S, o, 0.0)
        srow = jnp.sum(om, axis=0, keepdims=True)
        sqrow = jnp.sum(om * om, axis=0, keepdims=True)
        onehot = (_row_iota((E, H), 0) == t1).astype(jnp.float32)
        sums_ref[...] += onehot * srow
        sumsq_ref[...] += onehot * sqrow
        cnt_ref[...] += onehot


def _gcnB_body(x_ref, o_ref, rW_ref, rb_ref, sums_ref, sumsq_ref, cnt_ref,
               g_ref, bb_ref, predW_ref, predb_ref, prev_ref,
               xout_ref, tot_ref):
    rid = _row_iota((SP, 1), 0)
    for i in range(GPB):
        x = x_ref[i]
        scores = _dot(_seq_mean(x), rW_ref[...]) + rb_ref[...]
        t1 = _top1(scores)
        cnt = cnt_ref[pl.ds(t1, 1), :]
        cntS = jnp.maximum(cnt * jnp.float32(S), 1.0)
        mu = sums_ref[pl.ds(t1, 1), :] / cntS
        ex2 = sumsq_ref[pl.ds(t1, 1), :] / cntS
        var = ex2 - mu * mu
        o = o_ref[i]
        obn = (o - mu) * jax.lax.rsqrt(var + 1e-5) * g_ref[pl.ds(t1, 1), :] \
            + bb_ref[pl.ds(t1, 1), :]
        xn = jnp.maximum(obn, 0.0)
        xn = jnp.where(rid < S, xn, 0.0)
        xout_ref[i] = xn
        tot = prev_ref[i] + _dot(_seq_mean(xn), predW_ref[...]) + predb_ref[...]
        tot_ref[i] = tot


def _gcn(x, A, W, rW, rb, g, bb, predW, predb, prev):
    c = lambda shape: pl.BlockSpec(shape, lambda b: (0,) * len(shape))
    o, sums, sumsq, cnt = pl.pallas_call(
        _gcnA_body,
        grid=(B // GPB,),
        in_specs=[
            pl.BlockSpec((GPB, SP, H), lambda b: (b, 0, 0)),
            pl.BlockSpec((GPB, SP, SP), lambda b: (b, 0, 0)),
            c((E, H, H)), c((H, E)), c((1, E)),
        ],
        out_specs=[
            pl.BlockSpec((GPB, SP, H), lambda b: (b, 0, 0)),
            c((E, H)), c((E, H)), c((E, H)),
        ],
        out_shape=[
            jax.ShapeDtypeStruct((B, SP, H), jnp.float32),
            jax.ShapeDtypeStruct((E, H), jnp.float32),
            jax.ShapeDtypeStruct((E, H), jnp.float32),
            jax.ShapeDtypeStruct((E, H), jnp.float32),
        ],
        compiler_params=pltpu.CompilerParams(
            dimension_semantics=("arbitrary",)),
    )(x, A, W, rW, rb)
    xn, tot = pl.pallas_call(
        _gcnB_body,
        grid=(B // GPB,),
        in_specs=[
            pl.BlockSpec((GPB, SP, H), lambda b: (b, 0, 0)),
            pl.BlockSpec((GPB, SP, H), lambda b: (b, 0, 0)),
            c((H, E)), c((1, E)),
            c((E, H)), c((E, H)), c((E, H)),
            c((E, H)), c((E, H)),
            c((H, 128)), c((1, 128)),
            pl.BlockSpec((GPB, 1, 128), lambda b: (b, 0, 0)),
        ],
        out_specs=[
            pl.BlockSpec((GPB, SP, H), lambda b: (b, 0, 0)),
            pl.BlockSpec((GPB, 1, 128), lambda b: (b, 0, 0)),
        ],
        out_shape=[
            jax.ShapeDtypeStruct((B, SP, H), jnp.float32),
            jax.ShapeDtypeStruct((B, 1, 128), jnp.float32),
        ],
    )(x, o, rW, rb, sums, sumsq, cnt, g, bb, predW, predb, prev)
    return xn, tot


def kernel(node_features, Adj_block, node_prompt, parc_token, disease_emb,
           proj_W, proj_b, dis_W, dis_b, attn_Wqkv, attn_bqkv, attn_Wo,
           attn_bo, ln1_g, ln1_b, ln2_g, ln2_b, ffn_rW, ffn_rb, ffn_W1,
           ffn_b1, ffn_W2, ffn_b2, gcn_rW, gcn_rb, gcn_W, bn_g, bn_b,
           pred_W, pred_b):
    f32 = jnp.float32
    bf16 = jnp.bfloat16
    nf_pad = jnp.zeros((B, SP, MAXF), f32).at[:, 2:2 + N, :F].set(node_features)
    adj_shift = jnp.zeros((B, SP, SP), f32).at[:, 2:2 + N, 2:2 + N].set(Adj_block)
    adj_gcn = adj_shift.at[:, :2, :S].set(1.0).at[:, :S, :2].set(1.0).astype(bf16)
    prompt = jnp.zeros((1, SP, MAXF), f32).at[0, 2:2 + N, :].set(node_prompt[0, :N, :])
    predW_pad = jnp.zeros((G, H, 128), f32).at[:, :, :NC].set(pred_W).astype(bf16)
    predb_pad = jnp.zeros((G, 1, 128), f32).at[:, 0, :NC].set(pred_b)
    proj_W = proj_W.astype(bf16)
    dis_W = dis_W.astype(bf16)
    attn_Wqkv = attn_Wqkv.astype(bf16)
    attn_Wo = attn_Wo.astype(bf16)
    ffn_rW = ffn_rW.astype(bf16)
    ffn_W1 = ffn_W1.astype(bf16)
    ffn_W2 = ffn_W2.astype(bf16)
    gcn_rW = gcn_rW.astype(bf16)
    gcn_W = gcn_W.astype(bf16)

    x = _stage1(adj_shift, nf_pad, prompt, proj_W,
                proj_b.reshape(1, H), parc_token.reshape(1, MAXF),
                disease_emb.reshape(1, 768), dis_W, dis_b.reshape(1, H))

    tot = jnp.zeros((B, 1, 128), f32)
    for gl in range(G):
        for al in range(LP):
            l = gl * LP + al
            x = _layer(x, attn_Wqkv[l], attn_bqkv[l].reshape(1, 3 * H),
                       attn_Wo[l], attn_bo[l].reshape(1, H),
                       ln1_g[l].reshape(1, H), ln1_b[l].reshape(1, H),
                       ln2_g[l].reshape(1, H), ln2_b[l].reshape(1, H),
                       ffn_rW[l], ffn_rb[l].reshape(1, E),
                       ffn_W1[l], ffn_b1[l], ffn_W2[l], ffn_b2[l])
        x, tot = _gcn(x, adj_gcn, gcn_W[gl], gcn_rW[gl],
                      gcn_rb[gl].reshape(1, E), bn_g[gl], bn_b[gl],
                      predW_pad[gl], predb_pad[gl], tot)
    return tot[:, 0, :NC]


# speedup vs baseline: 1.9722x; 1.0030x over previous
import jax
import jax.numpy as jnp
from jax.experimental import pallas as pl
from jax.experimental.pallas import tpu as pltpu

B = 64
N = 200
F = 200
H = 256
MAXF = 256
RW = 5
NH = 8
E = 8
G = 2
LP = 2
DFF = 1024
NC = 2
S = N + 2
SP = 208
HD = H // NH
NEG = -1e30
GPB = 4


def _row_iota(shape, dim):
    return jax.lax.broadcasted_iota(jnp.int32, shape, dim)


def _ln(t, g, b):
    mu = jnp.mean(t, axis=-1, keepdims=True)
    var = jnp.mean((t - mu) ** 2, axis=-1, keepdims=True)
    return (t - mu) * jax.lax.rsqrt(var + 1e-5) * g + b


def _top1(scores):
    mx = jnp.max(scores, axis=-1, keepdims=True)
    ids = _row_iota(scores.shape, 1)
    cand = jnp.where(scores >= mx, ids, jnp.int32(E))
    return jnp.min(cand).astype(jnp.int32)


def _bf(a):
    return a if a.dtype == jnp.bfloat16 else a.astype(jnp.bfloat16)


def _dot(a, b):
    return jax.lax.dot_general(_bf(a), _bf(b),
                               (((a.ndim - 1,), (0,)), ((), ())),
                               preferred_element_type=jnp.float32)


def _dot_t(a, b):
    return jax.lax.dot_general(_bf(a), _bf(b),
                               (((1,), (1,)), ((), ())),
                               preferred_element_type=jnp.float32)


def _seq_mean(t):
    rows = _row_iota((SP, 1), 0)
    tm = jnp.where(rows < S, t, 0.0)
    return jnp.sum(tm, axis=0, keepdims=True) * (1.0 / S)


def _stage1_body(adj_ref, nf_ref, prompt_ref, projW_ref, projb_ref,
                 parc_ref, dis_ref, disW_ref, disb_ref, x_ref):
    rows = _row_iota((SP, SP), 0)
    cols = _row_iota((SP, SP), 1)
    eye = (rows == cols).astype(jnp.float32)
    rid = _row_iota((SP, 1), 0)
    node_row = jnp.logical_and(rid >= 2, rid < S)
    colf = _row_iota((SP, MAXF), 1)
    dis = _dot(dis_ref[...], disW_ref[...]) + disb_ref[...]
    parc = _dot(parc_ref[...], projW_ref[...]) + projb_ref[...]
    for i in range(GPB):
        a = adj_ref[i]
        adj = a / (jnp.sum(a, axis=-1, keepdims=True) + 1e-6)
        rw = adj
        diags = []
        for k in range(RW):
            diags.append(jnp.sum(rw * eye, axis=-1, keepdims=True))
            if k < RW - 1:
                rw = _dot(rw, adj)
        nf = nf_ref[i]
        for k in range(RW):
            nf = nf + jnp.where(colf == (F + k), diags[k], 0.0)
        nf = nf * prompt_ref[0]
        xp = _dot(nf, projW_ref[...]) + projb_ref[...]
        xp = jnp.where(node_row, xp, 0.0)
        xp = xp + jnp.where(rid == 0, dis, 0.0) + jnp.where(rid == 1, parc, 0.0)
        x_ref[i] = xp


def _stage1(adj_shift, nf_pad, prompt, projW, projb, parc, dis, disW, disb):
    return pl.pallas_call(
        _stage1_body,
        grid=(B // GPB,),
        in_specs=[
            pl.BlockSpec((GPB, SP, SP), lambda b: (b, 0, 0)),
            pl.BlockSpec((GPB, SP, MAXF), lambda b: (b, 0, 0)),
            pl.BlockSpec((1, SP, MAXF), lambda b: (0, 0, 0)),
            pl.BlockSpec((MAXF, H), lambda b: (0, 0)),
            pl.BlockSpec((1, H), lambda b: (0, 0)),
            pl.BlockSpec((1, MAXF), lambda b: (0, 0)),
            pl.BlockSpec((1, 768), lambda b: (0, 0)),
            pl.BlockSpec((768, H), lambda b: (0, 0)),
            pl.BlockSpec((1, H), lambda b: (0, 0)),
        ],
        out_specs=pl.BlockSpec((GPB, SP, H), lambda b: (b, 0, 0)),
        out_shape=jax.ShapeDtypeStruct((B, SP, H), jnp.float32),
        compiler_params=pltpu.CompilerParams(
            dimension_semantics=("parallel",)),
    )(adj_shift, nf_pad, prompt, projW, projb, parc, dis, disW, disb)


def _layer_body(x_ref, Wqkv_ref, bqkv_ref, Wo_ref, bo_ref, g1_ref, b1_ref,
                g2_ref, b2_ref, rW_ref, rb_ref, W1_ref, fb1_ref, W2_ref,
                fb2_ref, out_ref):
    rid = _row_iota((SP, 1), 0)
    scale = 1.0 / (HD ** 0.5)
    ones_col = jnp.ones((SP, 1), jnp.float32)
    for i in range(GPB):
        x = x_ref[i]
        qkv = _dot(x, Wqkv_ref[...]) + bqkv_ref[...]
        outs = []
        for h in range(NH):
            qh = qkv[:, h * HD:(h + 1) * HD] * scale
            kh = qkv[:, H + h * HD:H + (h + 1) * HD]
            vh = qkv[:, 2 * H + h * HD:2 * H + (h + 1) * HD]
            s = _dot_t(qh, kh)
            e = jnp.exp(s)
            oe = _dot(e, jnp.concatenate([vh, ones_col], axis=-1))
            outs.append(oe[:, :HD] / (oe[:, HD:HD + 1] - (SP - S)))
        o = jnp.concatenate(outs, axis=-1)
        a = _dot(o, Wo_ref[...]) + bo_ref[...]
        x = _ln(x + a, g1_ref[...], b1_ref[...])
        scores = _dot(_seq_mean(x), rW_ref[...]) + rb_ref[...]
        t1 = _top1(scores)
        w1 = W1_ref[t1]
        b1v = fb1_ref[pl.ds(t1, 1), :]
        w2 = W2_ref[t1]
        b2v = fb2_ref[pl.ds(t1, 1), :]
        hdn = jnp.maximum(_dot(x, w1) + b1v, 0.0)
        y = _dot(hdn, w2) + b2v
        x = _ln(x + y, g2_ref[...], b2_ref[...])
        out_ref[i] = jnp.where(rid < S, x, 0.0)


def _layer(x, Wqkv, bqkv, Wo, bo, g1, b1, g2, b2, rW, rb, W1, fb1, W2, fb2):
    c = lambda shape: pl.BlockSpec(shape, lambda b: (0,) * len(shape))
    return pl.pallas_call(
        _layer_body,
        grid=(B // GPB,),
        in_specs=[
            pl.BlockSpec((GPB, SP, H), lambda b: (b, 0, 0)),
            c((H, 3 * H)), c((1, 3 * H)), c((H, H)), c((1, H)),
            c((1, H)), c((1, H)), c((1, H)), c((1, H)),
            c((H, E)), c((1, E)),
            c((E, H, DFF)), c((E, DFF)), c((E, DFF, H)), c((E, H)),
        ],
        out_specs=pl.BlockSpec((GPB, SP, H), lambda b: (b, 0, 0)),
        out_shape=jax.ShapeDtypeStruct((B, SP, H), jnp.float32),
        compiler_params=pltpu.CompilerParams(
            dimension_semantics=("parallel",)),
    )(x, Wqkv, bqkv, Wo, bo, g1, b1, g2, b2, rW, rb, W1, fb1, W2, fb2)


def _gcnA_body(x_ref, A_ref, W_ref, rW_ref, rb_ref,
               o_ref, sums_ref, sumsq_ref, cnt_ref):
    b = pl.program_id(0)

    @pl.when(b == 0)
    def _init():
        sums_ref[...] = jnp.zeros((E, H), jnp.float32)
        sumsq_ref[...] = jnp.zeros((E, H), jnp.float32)
        cnt_ref[...] = jnp.zeros((E, H), jnp.float32)

    rid = _row_iota((SP, 1), 0)
    for i in range(GPB):
        x = x_ref[i]
        scores = _dot(_seq_mean(x), rW_ref[...]) + rb_ref[...]
        t1 = _top1(scores)
        w = W_ref[t1]
        o = _dot(A_ref[i], _dot(x, w))
        o_ref[i] = o
        om = jnp.where(rid < S, o, 0.0)
        srow = jnp.sum(om, axis=0, keepdims=True)
        sqrow = jnp.sum(om * om, axis=0, keepdims=True)
        onehot = (_row_iota((E, H), 0) == t1).astype(jnp.float32)
        sums_ref[...] += onehot * srow
        sumsq_ref[...] += onehot * sqrow
        cnt_ref[...] += onehot


def _gcnB_body(x_ref, o_ref, rW_ref, rb_ref, sums_ref, sumsq_ref, cnt_ref,
               g_ref, bb_ref, predW_ref, predb_ref, prev_ref,
               xout_ref, tot_ref):
    rid = _row_iota((SP, 1), 0)
    for i in range(GPB):
        x = x_ref[i]
        scores = _dot(_seq_mean(x), rW_ref[...]) + rb_ref[...]
        t1 = _top1(scores)
        cnt = cnt_ref[pl.ds(t1, 1), :]
        cntS = jnp.maximum(cnt * jnp.float32(S), 1.0)
        mu = sums_ref[pl.ds(t1, 1), :] / cntS
        ex2 = sumsq_ref[pl.ds(t1, 1), :] / cntS
        var = ex2 - mu * mu
        o = o_ref[i]
        obn = (o - mu) * jax.lax.rsqrt(var + 1e-5) * g_ref[pl.ds(t1, 1), :] \
            + bb_ref[pl.ds(t1, 1), :]
        xn = jnp.maximum(obn, 0.0)
        xn = jnp.where(rid < S, xn, 0.0)
        xout_ref[i] = xn
        tot = prev_ref[i] + _dot(_seq_mean(xn), predW_ref[...]) + predb_ref[...]
        tot_ref[i] = tot


def _gcn(x, A, W, rW, rb, g, bb, predW, predb, prev):
    c = lambda shape: pl.BlockSpec(shape, lambda b: (0,) * len(shape))
    o, sums, sumsq, cnt = pl.pallas_call(
        _gcnA_body,
        grid=(B // GPB,),
        in_specs=[
            pl.BlockSpec((GPB, SP, H), lambda b: (b, 0, 0)),
            pl.BlockSpec((GPB, SP, SP), lambda b: (b, 0, 0)),
            c((E, H, H)), c((H, E)), c((1, E)),
        ],
        out_specs=[
            pl.BlockSpec((GPB, SP, H), lambda b: (b, 0, 0)),
            c((E, H)), c((E, H)), c((E, H)),
        ],
        out_shape=[
            jax.ShapeDtypeStruct((B, SP, H), jnp.float32),
            jax.ShapeDtypeStruct((E, H), jnp.float32),
            jax.ShapeDtypeStruct((E, H), jnp.float32),
            jax.ShapeDtypeStruct((E, H), jnp.float32),
        ],
        compiler_params=pltpu.CompilerParams(
            dimension_semantics=("arbitrary",)),
    )(x, A, W, rW, rb)
    xn, tot = pl.pallas_call(
        _gcnB_body,
        grid=(B // GPB,),
        in_specs=[
            pl.BlockSpec((GPB, SP, H), lambda b: (b, 0, 0)),
            pl.BlockSpec((GPB, SP, H), lambda b: (b, 0, 0)),
            c((H, E)), c((1, E)),
            c((E, H)), c((E, H)), c((E, H)),
            c((E, H)), c((E, H)),
            c((H, 128)), c((1, 128)),
            pl.BlockSpec((GPB, 1, 128), lambda b: (b, 0, 0)),
        ],
        out_specs=[
            pl.BlockSpec((GPB, SP, H), lambda b: (b, 0, 0)),
            pl.BlockSpec((GPB, 1, 128), lambda b: (b, 0, 0)),
        ],
        out_shape=[
            jax.ShapeDtypeStruct((B, SP, H), jnp.float32),
            jax.ShapeDtypeStruct((B, 1, 128), jnp.float32),
        ],
        compiler_params=pltpu.CompilerParams(
            dimension_semantics=("parallel",)),
    )(x, o, rW, rb, sums, sumsq, cnt, g, bb, predW, predb, prev)
    return xn, tot


def kernel(node_features, Adj_block, node_prompt, parc_token, disease_emb,
           proj_W, proj_b, dis_W, dis_b, attn_Wqkv, attn_bqkv, attn_Wo,
           attn_bo, ln1_g, ln1_b, ln2_g, ln2_b, ffn_rW, ffn_rb, ffn_W1,
           ffn_b1, ffn_W2, ffn_b2, gcn_rW, gcn_rb, gcn_W, bn_g, bn_b,
           pred_W, pred_b):
    f32 = jnp.float32
    bf16 = jnp.bfloat16
    nf_pad = jnp.zeros((B, SP, MAXF), f32).at[:, 2:2 + N, :F].set(node_features)
    adj_shift = jnp.zeros((B, SP, SP), f32).at[:, 2:2 + N, 2:2 + N].set(Adj_block)
    adj_gcn = adj_shift.at[:, :2, :S].set(1.0).at[:, :S, :2].set(1.0).astype(bf16)
    prompt = jnp.zeros((1, SP, MAXF), f32).at[0, 2:2 + N, :].set(node_prompt[0, :N, :])
    predW_pad = jnp.zeros((G, H, 128), f32).at[:, :, :NC].set(pred_W).astype(bf16)
    predb_pad = jnp.zeros((G, 1, 128), f32).at[:, 0, :NC].set(pred_b)
    proj_W = proj_W.astype(bf16)
    dis_W = dis_W.astype(bf16)
    attn_Wqkv = attn_Wqkv.astype(bf16)
    attn_Wo = attn_Wo.astype(bf16)
    ffn_rW = ffn_rW.astype(bf16)
    ffn_W1 = ffn_W1.astype(bf16)
    ffn_W2 = ffn_W2.astype(bf16)
    gcn_rW = gcn_rW.astype(bf16)
    gcn_W = gcn_W.astype(bf16)

    x = _stage1(adj_shift, nf_pad, prompt, proj_W,
                proj_b.reshape(1, H), parc_token.reshape(1, MAXF),
                disease_emb.reshape(1, 768), dis_W, dis_b.reshape(1, H))

    tot = jnp.zeros((B, 1, 128), f32)
    for gl in range(G):
        for al in range(LP):
            l = gl * LP + al
            x = _layer(x, attn_Wqkv[l], attn_bqkv[l].reshape(1, 3 * H),
                       attn_Wo[l], attn_bo[l].reshape(1, H),
                       ln1_g[l].reshape(1, H), ln1_b[l].reshape(1, H),
                       ln2_g[l].reshape(1, H), ln2_b[l].reshape(1, H),
                       ffn_rW[l], ffn_rb[l].reshape(1, E),
                       ffn_W1[l], ffn_b1[l], ffn_W2[l], ffn_b2[l])
        x, tot = _gcn(x, adj_gcn, gcn_W[gl], gcn_rW[gl],
                      gcn_rb[gl].reshape(1, E), bn_g[gl], bn_b[gl],
                      predW_pad[gl], predb_pad[gl], tot)
    return tot[:, 0, :NC]


# speedup vs baseline: 1.9868x; 1.0074x over previous
import jax
import jax.numpy as jnp
from jax.experimental import pallas as pl
from jax.experimental.pallas import tpu as pltpu

B = 64
N = 200
F = 200
H = 256
MAXF = 256
RW = 5
NH = 8
E = 8
G = 2
LP = 2
DFF = 1024
NC = 2
L = G * LP
S = N + 2
SP = 208
HD = H // NH
NEG = -1e30
GPB = 8
HDE = HD + 1
WVE = NH * HDE


def _row_iota(shape, dim):
    return jax.lax.broadcasted_iota(jnp.int32, shape, dim)


def _ln(t, g, b):
    mu = jnp.mean(t, axis=-1, keepdims=True)
    var = jnp.mean((t - mu) ** 2, axis=-1, keepdims=True)
    return (t - mu) * jax.lax.rsqrt(var + 1e-5) * g + b


def _top1(scores):
    mx = jnp.max(scores, axis=-1, keepdims=True)
    ids = _row_iota(scores.shape, 1)
    cand = jnp.where(scores >= mx, ids, jnp.int32(E))
    return jnp.min(cand).astype(jnp.int32)


def _bf(a):
    return a if a.dtype == jnp.bfloat16 else a.astype(jnp.bfloat16)


def _dot(a, b):
    return jax.lax.dot_general(_bf(a), _bf(b),
                               (((a.ndim - 1,), (0,)), ((), ())),
                               preferred_element_type=jnp.float32)


def _dot_t(a, b):
    return jax.lax.dot_general(_bf(a), _bf(b),
                               (((1,), (1,)), ((), ())),
                               preferred_element_type=jnp.float32)


def _seq_mean(t):
    rows = _row_iota((SP, 1), 0)
    tm = jnp.where(rows < S, t, 0.0)
    return jnp.sum(tm, axis=0, keepdims=True) * (1.0 / S)


def _stage1_body(adj_ref, nf_ref, prompt_ref, projW_ref, projb_ref,
                 parc_ref, dis_ref, disW_ref, disb_ref, x_ref):
    rows = _row_iota((SP, SP), 0)
    cols = _row_iota((SP, SP), 1)
    eye = (rows == cols).astype(jnp.float32)
    rid = _row_iota((SP, 1), 0)
    node_row = jnp.logical_and(rid >= 2, rid < S)
    colf = _row_iota((SP, MAXF), 1)
    dis = _dot(dis_ref[...], disW_ref[...]) + disb_ref[...]
    parc = _dot(parc_ref[...], projW_ref[...]) + projb_ref[...]
    for i in range(GPB):
        a = adj_ref[i]
        adj = a / (jnp.sum(a, axis=-1, keepdims=True) + 1e-6)
        rw = adj
        diags = []
        for k in range(RW):
            diags.append(jnp.sum(rw * eye, axis=-1, keepdims=True))
            if k < RW - 1:
                rw = _dot(rw, adj)
        nf = nf_ref[i]
        for k in range(RW):
            nf = nf + jnp.where(colf == (F + k), diags[k], 0.0)
        nf = nf * prompt_ref[0]
        xp = _dot(nf, projW_ref[...]) + projb_ref[...]
        xp = jnp.where(node_row, xp, 0.0)
        xp = xp + jnp.where(rid == 0, dis, 0.0) + jnp.where(rid == 1, parc, 0.0)
        x_ref[i] = xp


def _stage1(adj_shift, nf_pad, prompt, projW, projb, parc, dis, disW, disb):
    return pl.pallas_call(
        _stage1_body,
        grid=(B // GPB,),
        in_specs=[
            pl.BlockSpec((GPB, SP, SP), lambda b: (b, 0, 0)),
            pl.BlockSpec((GPB, SP, MAXF), lambda b: (b, 0, 0)),
            pl.BlockSpec((1, SP, MAXF), lambda b: (0, 0, 0)),
            pl.BlockSpec((MAXF, H), lambda b: (0, 0)),
            pl.BlockSpec((1, H), lambda b: (0, 0)),
            pl.BlockSpec((1, MAXF), lambda b: (0, 0)),
            pl.BlockSpec((1, 768), lambda b: (0, 0)),
            pl.BlockSpec((768, H), lambda b: (0, 0)),
            pl.BlockSpec((1, H), lambda b: (0, 0)),
        ],
        out_specs=pl.BlockSpec((GPB, SP, H), lambda b: (b, 0, 0)),
        out_shape=jax.ShapeDtypeStruct((B, SP, H), jnp.float32),
    )(adj_shift, nf_pad, prompt, projW, projb, parc, dis, disW, disb)


def _layer_body(x_ref, Wqk_ref, bqk_ref, Wve_ref, bve_ref, Wo_ref, bo_ref,
                g1_ref, b1_ref, g2_ref, b2_ref, rW_ref, rb_ref, W1_ref,
                fb1_ref, W2_ref, fb2_ref, out_ref):
    rid = _row_iota((SP, 1), 0)
    scale = 1.0 / (HD ** 0.5)
    for i in range(GPB):
        x = x_ref[i]
        qk = _dot(x, Wqk_ref[...]) + bqk_ref[...]
        vf = _dot(x, Wve_ref[...]) + bve_ref[...]
        outs = []
        for h in range(NH):
            qh = qk[:, h * HD:(h + 1) * HD] * scale
            kh = qk[:, H + h * HD:H + (h + 1) * HD]
            ve = vf[:, h * HDE:(h + 1) * HDE]
            s = _dot_t(qh, kh)
            e = jnp.exp(s)
            oe = _dot(e, ve)
            outs.append(oe[:, :HD] / (oe[:, HD:HD + 1] - (SP - S)))
        o = jnp.concatenate(outs, axis=-1)
        a = _dot(o, Wo_ref[...]) + bo_ref[...]
        x = _ln(x + a, g1_ref[...], b1_ref[...])
        scores = _dot(_seq_mean(x), rW_ref[...]) + rb_ref[...]
        t1 = _top1(scores)
        w1 = W1_ref[t1]
        b1v = fb1_ref[pl.ds(t1, 1), :]
        w2 = W2_ref[t1]
        b2v = fb2_ref[pl.ds(t1, 1), :]
        hdn = jnp.maximum(_dot(x, w1) + b1v, 0.0)
        y = _dot(hdn, w2) + b2v
        x = _ln(x + y, g2_ref[...], b2_ref[...])
        out_ref[i] = jnp.where(rid < S, x, 0.0)


def _layer(x, Wqk, bqk, Wve, bve, Wo, bo, g1, b1, g2, b2, rW, rb,
           W1, fb1, W2, fb2):
    c = lambda shape: pl.BlockSpec(shape, lambda b: (0,) * len(shape))
    return pl.pallas_call(
        _layer_body,
        grid=(B // GPB,),
        in_specs=[
            pl.BlockSpec((GPB, SP, H), lambda b: (b, 0, 0)),
            c((H, 2 * H)), c((1, 2 * H)), c((H, WVE)), c((1, WVE)),
            c((H, H)), c((1, H)),
            c((1, H)), c((1, H)), c((1, H)), c((1, H)),
            c((H, E)), c((1, E)),
            c((E, H, DFF)), c((E, DFF)), c((E, DFF, H)), c((E, H)),
        ],
        out_specs=pl.BlockSpec((GPB, SP, H), lambda b: (b, 0, 0)),
        out_shape=jax.ShapeDtypeStruct((B, SP, H), jnp.float32),
    )(x, Wqk, bqk, Wve, bve, Wo, bo, g1, b1, g2, b2, rW, rb, W1, fb1, W2, fb2)


def _gcnA_body(x_ref, A_ref, W_ref, rW_ref, rb_ref,
               o_ref, sums_ref, sumsq_ref, cnt_ref):
    b = pl.program_id(0)

    @pl.when(b == 0)
    def _init():
        sums_ref[...] = jnp.zeros((E, H), jnp.float32)
        sumsq_ref[...] = jnp.zeros((E, H), jnp.float32)
        cnt_ref[...] = jnp.zeros((E, H), jnp.float32)

    rid = _row_iota((SP, 1), 0)
    for i in range(GPB):
        x = x_ref[i]
        scores = _dot(_seq_mean(x), rW_ref[...]) + rb_ref[...]
        t1 = _top1(scores)
        w = W_ref[t1]
        o = _dot(A_ref[i], _dot(x, w))
        o_ref[i] = o
        om = jnp.where(rid < S, o, 0.0)
        srow = jnp.sum(om, axis=0, keepdims=True)
        sqrow = jnp.sum(om * om, axis=0, keepdims=True)
        onehot = (_row_iota((E, H), 0) == t1).astype(jnp.float32)
        sums_ref[...] += onehot * srow
        sumsq_ref[...] += onehot * sqrow
        cnt_ref[...] += onehot


def _gcnB_body(x_ref, o_ref, rW_ref, rb_ref, sums_ref, sumsq_ref, cnt_ref,
               g_ref, bb_ref, predW_ref, predb_ref, prev_ref,
               xout_ref, tot_ref):
    rid = _row_iota((SP, 1), 0)
    for i in range(GPB):
        x = x_ref[i]
        scores = _dot(_seq_mean(x), rW_ref[...]) + rb_ref[...]
        t1 = _top1(scores)
        cnt = cnt_ref[pl.ds(t1, 1), :]
        cntS = jnp.maximum(cnt * jnp.float32(S), 1.0)
        mu = sums_ref[pl.ds(t1, 1), :] / cntS
        ex2 = sumsq_ref[pl.ds(t1, 1), :] / cntS
        var = ex2 - mu * mu
        o = o_ref[i]
        obn = (o - mu) * jax.lax.rsqrt(var + 1e-5) * g_ref[pl.ds(t1, 1), :] \
            + bb_ref[pl.ds(t1, 1), :]
        xn = jnp.maximum(obn, 0.0)
        xn = jnp.where(rid < S, xn, 0.0)
        xout_ref[i] = xn
        tot = prev_ref[i] + _dot(_seq_mean(xn), predW_ref[...]) + predb_ref[...]
        tot_ref[i] = tot


def _gcn(x, A, W, rW, rb, g, bb, predW, predb, prev):
    c = lambda shape: pl.BlockSpec(shape, lambda b: (0,) * len(shape))
    o, sums, sumsq, cnt = pl.pallas_call(
        _gcnA_body,
        grid=(B // GPB,),
        in_specs=[
            pl.BlockSpec((GPB, SP, H), lambda b: (b, 0, 0)),
            pl.BlockSpec((GPB, SP, SP), lambda b: (b, 0, 0)),
            c((E, H, H)), c((H, E)), c((1, E)),
        ],
        out_specs=[
            pl.BlockSpec((GPB, SP, H), lambda b: (b, 0, 0)),
            c((E, H)), c((E, H)), c((E, H)),
        ],
        out_shape=[
            jax.ShapeDtypeStruct((B, SP, H), jnp.float32),
            jax.ShapeDtypeStruct((E, H), jnp.float32),
            jax.ShapeDtypeStruct((E, H), jnp.float32),
            jax.ShapeDtypeStruct((E, H), jnp.float32),
        ],
        compiler_params=pltpu.CompilerParams(
            dimension_semantics=("arbitrary",)),
    )(x, A, W, rW, rb)
    xn, tot = pl.pallas_call(
        _gcnB_body,
        grid=(B // GPB,),
        in_specs=[
            pl.BlockSpec((GPB, SP, H), lambda b: (b, 0, 0)),
            pl.BlockSpec((GPB, SP, H), lambda b: (b, 0, 0)),
            c((H, E)), c((1, E)),
            c((E, H)), c((E, H)), c((E, H)),
            c((E, H)), c((E, H)),
            c((H, 128)), c((1, 128)),
            pl.BlockSpec((GPB, 1, 128), lambda b: (b, 0, 0)),
        ],
        out_specs=[
            pl.BlockSpec((GPB, SP, H), lambda b: (b, 0, 0)),
            pl.BlockSpec((GPB, 1, 128), lambda b: (b, 0, 0)),
        ],
        out_shape=[
            jax.ShapeDtypeStruct((B, SP, H), jnp.float32),
            jax.ShapeDtypeStruct((B, 1, 128), jnp.float32),
        ],
    )(x, o, rW, rb, sums, sumsq, cnt, g, bb, predW, predb, prev)
    return xn, tot


def kernel(node_features, Adj_block, node_prompt, parc_token, disease_emb,
           proj_W, proj_b, dis_W, dis_b, attn_Wqkv, attn_bqkv, attn_Wo,
           attn_bo, ln1_g, ln1_b, ln2_g, ln2_b, ffn_rW, ffn_rb, ffn_W1,
           ffn_b1, ffn_W2, ffn_b2, gcn_rW, gcn_rb, gcn_W, bn_g, bn_b,
           pred_W, pred_b):
    f32 = jnp.float32
    bf16 = jnp.bfloat16
    nf_pad = jnp.zeros((B, SP, MAXF), f32).at[:, 2:2 + N, :F].set(node_features)
    adj_shift = jnp.zeros((B, SP, SP), f32).at[:, 2:2 + N, 2:2 + N].set(Adj_block)
    adj_gcn = adj_shift.at[:, :2, :S].set(1.0).at[:, :S, :2].set(1.0).astype(bf16)
    prompt = jnp.zeros((1, SP, MAXF), f32).at[0, 2:2 + N, :].set(node_prompt[0, :N, :])
    predW_pad = jnp.zeros((G, H, 128), f32).at[:, :, :NC].set(pred_W).astype(bf16)
    predb_pad = jnp.zeros((G, 1, 128), f32).at[:, 0, :NC].set(pred_b)
    proj_W = proj_W.astype(bf16)
    dis_W = dis_W.astype(bf16)
    attn_Wqk = attn_Wqkv[:, :, :2 * H].astype(bf16)
    attn_bqk = attn_bqkv[:, :2 * H].reshape(L, 1, 2 * H)
    Wv = attn_Wqkv[:, :, 2 * H:].reshape(L, H, NH, HD)
    attn_Wve = jnp.concatenate(
        [Wv, jnp.zeros((L, H, NH, 1), f32)], axis=-1).reshape(
            L, H, WVE).astype(bf16)
    bv = attn_bqkv[:, 2 * H:].reshape(L, NH, HD)
    attn_bve = jnp.concatenate(
        [bv, jnp.ones((L, NH, 1), f32)], axis=-1).reshape(L, 1, WVE)
    attn_Wo = attn_Wo.astype(bf16)
    ffn_rW = ffn_rW.astype(bf16)
    ffn_W1 = ffn_W1.astype(bf16)
    ffn_W2 = ffn_W2.astype(bf16)
    gcn_rW = gcn_rW.astype(bf16)
    gcn_W = gcn_W.astype(bf16)

    x = _stage1(adj_shift, nf_pad, prompt, proj_W,
                proj_b.reshape(1, H), parc_token.reshape(1, MAXF),
                disease_emb.reshape(1, 768), dis_W, dis_b.reshape(1, H))

    tot = jnp.zeros((B, 1, 128), f32)
    for gl in range(G):
        for al in range(LP):
            l = gl * LP + al
            x = _layer(x, attn_Wqk[l], attn_bqk[l], attn_Wve[l],
                       attn_bve[l],
                       attn_Wo[l], attn_bo[l].reshape(1, H),
                       ln1_g[l].reshape(1, H), ln1_b[l].reshape(1, H),
                       ln2_g[l].reshape(1, H), ln2_b[l].reshape(1, H),
                       ffn_rW[l], ffn_rb[l].reshape(1, E),
                       ffn_W1[l], ffn_b1[l], ffn_W2[l], ffn_b2[l])
        x, tot = _gcn(x, adj_gcn, gcn_W[gl], gcn_rW[gl],
                      gcn_rb[gl].reshape(1, E), bn_g[gl], bn_b[gl],
                      predW_pad[gl], predb_pad[gl], tot)
    return tot[:, 0, :NC]
